# baseline (speedup 1.0000x reference)
import functools

import jax
import jax.numpy as jnp
from jax import lax
from jax.experimental import pallas as pl
from jax.experimental.pallas import tpu as pltpu
from jax.experimental.pallas import tpu_sc as plsc

B = 4096
L = 200
D = 64
C = 2
V = 1000000

NC = 2
NS = 16
NW = NC * NS
BPW = B // NW
NBUF = 4

VP = 1 << 20
TBLK = 32768
TGRID = VP // TBLK
VPS = VP // NS


def _tc_project_body(w_ref, t_ref, o_ref):
    p = lax.dot_general(
        w_ref[...], t_ref[...], (((1,), (0,)), ((), ())),
        preferred_element_type=jnp.float32) * (1.0 / L)
    u0 = lax.bitcast_convert_type(
        p[0, :].astype(jnp.bfloat16), jnp.uint16).astype(jnp.uint32)
    u1 = lax.bitcast_convert_type(
        p[1, :].astype(jnp.bfloat16), jnp.uint16).astype(jnp.uint32)
    o_ref[...] = lax.bitcast_convert_type(u0 | (u1 << 16), jnp.float32)


_tc_project = pl.pallas_call(
    _tc_project_body,
    grid=(TGRID,),
    in_specs=[pl.BlockSpec((C, D), lambda i: (0, 0)),
              pl.BlockSpec((D, TBLK), lambda i: (0, jnp.minimum(i, TGRID - 2)))],
    out_specs=pl.BlockSpec((TBLK,), lambda i: (i,)),
    out_shape=jax.ShapeDtypeStruct((VP,), jnp.float32),
)

_mesh = plsc.VectorSubcoreMesh(core_axis_name="c", subcore_axis_name="s")


@functools.partial(
    pl.kernel,
    out_type=jax.ShapeDtypeStruct((C, B), jnp.float32),
    mesh=_mesh,
    compiler_params=pltpu.CompilerParams(use_tc_tiling_on_sc=False),
    scratch_types=[
        pltpu.VMEM((L, BPW), jnp.int32),
        pltpu.VMEM((NBUF, BPW), jnp.float32),
        pltpu.VMEM((C, BPW), jnp.float32),
        pltpu.VMEM((C, 16), jnp.float32),
        pltpu.VMEM_SHARED((VP,), jnp.float32),
        pltpu.SemaphoreType.DMA,
        pltpu.SemaphoreType.DMA,
        pltpu.SemaphoreType.DMA,
        pltpu.SemaphoreType.DMA,
    ],
)
def _sc_pool(xt_hbm, tw_hbm, bb_hbm, out_hbm, idx_v, gbuf, out_v, b_v,
             tw_s, sem0, sem1, sem2, sem3):
    sems = (sem0, sem1, sem2, sem3)
    sid = lax.axis_index("s")
    wid = sid * NC + lax.axis_index("c")
    base = wid * BPW

    off = pl.multiple_of(sid * VPS, 8)
    c_tw = pltpu.async_copy(tw_hbm.at[pl.ds(off, VPS)], tw_s.at[pl.ds(off, VPS)], sem0)
    c_ix = pltpu.async_copy(xt_hbm.at[:, pl.ds(base, BPW)], idx_v, sem1)
    pltpu.sync_copy(bb_hbm, b_v)
    c_tw.wait()
    c_ix.wait()
    plsc.subcore_barrier()

    def fire(j, p):
        pltpu.async_copy(tw_s.at[idx_v.at[j]], gbuf.at[p], sems[p])

    def wait(p):
        pltpu.make_async_copy(tw_s.at[pl.ds(0, BPW)], gbuf.at[p],
                              sems[p]).wait()

    for p in range(NBUF - 1):
        fire(p, p)

    hi = jnp.uint32(0xFFFF0000)

    def outer(i, accs):
        accs = list(accs)
        for p in range(NBUF):
            j = NBUF * i + p
            wait(p)
            for g in range(8):
                v = lax.bitcast_convert_type(
                    gbuf[p, pl.ds(16 * g, 16)], jnp.uint32)
                accs[g] = accs[g] + lax.bitcast_convert_type(
                    v << 16, jnp.float32)
                accs[8 + g] = accs[8 + g] + lax.bitcast_convert_type(
                    v & hi, jnp.float32)

            @pl.when(j + NBUF - 1 < L)
            def _():
                fire(j + NBUF - 1, (p + NBUF - 1) % NBUF)
        return tuple(accs)

    accs = lax.fori_loop(
        0, L // NBUF, outer, (jnp.zeros((16,), jnp.float32),) * 16)

    for g in range(8):
        out_v[0, pl.ds(16 * g, 16)] = accs[g] + b_v[0, :]
        out_v[1, pl.ds(16 * g, 16)] = accs[8 + g] + b_v[1, :]
    pltpu.sync_copy(out_v, out_hbm.at[:, pl.ds(base, BPW)])


def kernel(x, table, W, b):
    tw = _tc_project(W, table.T)
    bb = jnp.tile(b[:, None], (1, 16))
    out2 = _sc_pool(x.T, tw, bb)
    return out2.T
